# Initial kernel scaffold; baseline (speedup 1.0000x reference)
import jax
import jax.numpy as jnp
from jax.experimental import pallas as pl


def kernel(vert, W, a_src, a_dst):
    raise NotImplementedError("write your pallas kernel here")


# speedup vs baseline: 1.1065x; 1.1065x over previous
import jax
import jax.numpy as jnp
from jax.experimental import pallas as pl

N = 4096
IN_F = 512
OUT_F = 256
H = 4
HID = OUT_F // H

BM = 512
BI = 512
BJ = 512


def _proj_kernel(vert_ref, w_ref, acomb_ref, g_ref, ss_ref):
    g = jnp.dot(vert_ref[...], w_ref[...], preferred_element_type=jnp.float32)
    g_ref[...] = g
    ss_ref[...] = jnp.dot(g, acomb_ref[...], preferred_element_type=jnp.float32)


def _attn_kernel(ssrc_ref, sdst_row_ref, sdst_col_ref, tmax_ref, g_ref, out_ref):
    c = tmax_ref[0, 0, 0]
    s_col = ssrc_ref[0]
    x = s_col + c
    a_scl = jnp.exp(0.8 * jnp.minimum(x, 0.0))
    b_scl = jnp.exp(-0.8 * jnp.maximum(x, 0.0))

    acc = jnp.zeros((BI, 2 * HID), jnp.float32)
    accp = jnp.zeros((BI, 1), jnp.float32)
    accq = jnp.zeros((BI, 1), jnp.float32)
    qg_tot = jnp.zeros((1, HID), jnp.float32)
    q_tot = jnp.zeros((1, 1), jnp.float32)

    for jc in range(N // BJ):
        t_row = sdst_row_ref[0][:, jc * BJ:(jc + 1) * BJ]
        t_col = sdst_col_ref[0][jc * BJ:(jc + 1) * BJ, :]
        gj = g_ref[0][jc * BJ:(jc + 1) * BJ, :]
        p_row = jnp.exp(t_row - c)
        q_row = jnp.exp(0.2 * (t_row - c))
        p_col = jnp.exp(t_col - c)
        q_col = jnp.exp(0.2 * (t_col - c))
        cmat = jnp.concatenate([p_col * gj, q_col * gj], axis=1)
        mask = ((s_col + t_row) > 0.0).astype(jnp.float32)
        acc = acc + jnp.dot(mask, cmat, preferred_element_type=jnp.float32)
        accp = accp + jnp.sum(mask * p_row, axis=1, keepdims=True)
        accq = accq + jnp.sum(mask * q_row, axis=1, keepdims=True)
        qg_tot = qg_tot + jnp.sum(q_col * gj, axis=0, keepdims=True)
        q_tot = q_tot + jnp.sum(q_row, axis=1, keepdims=True)

    numer = a_scl * acc[:, :HID] + b_scl * (qg_tot - acc[:, HID:])
    denom = a_scl * accp + b_scl * (q_tot - accq)
    o = numer / denom
    out_ref[0] = jnp.where(o > 0.0, o, jnp.exp(jnp.minimum(o, 0.0)) - 1.0)


def kernel(vert, W, a_src, a_dst):
    idx = jnp.arange(OUT_F)
    head = idx // HID
    sel = (head[:, None] == jnp.arange(H)[None, :]).astype(jnp.float32)
    acomb = jnp.concatenate(
        [sel * a_src.reshape(-1)[:, None], sel * a_dst.reshape(-1)[:, None]],
        axis=1)

    g, ss = pl.pallas_call(
        _proj_kernel,
        grid=(N // BM,),
        in_specs=[
            pl.BlockSpec((BM, IN_F), lambda i: (i, 0)),
            pl.BlockSpec((IN_F, OUT_F), lambda i: (0, 0)),
            pl.BlockSpec((OUT_F, 2 * H), lambda i: (0, 0)),
        ],
        out_specs=[
            pl.BlockSpec((BM, OUT_F), lambda i: (i, 0)),
            pl.BlockSpec((BM, 2 * H), lambda i: (i, 0)),
        ],
        out_shape=[
            jax.ShapeDtypeStruct((N, OUT_F), jnp.float32),
            jax.ShapeDtypeStruct((N, 2 * H), jnp.float32),
        ],
    )(vert, W, acomb)

    ssrc = ss[:, :H].T
    sdst = ss[:, H:].T
    ssrc_col = ssrc.reshape(H, N, 1)
    sdst_row = sdst.reshape(H, 1, N)
    sdst_col = sdst.reshape(H, N, 1)
    tmax = jnp.max(sdst, axis=1).reshape(H, 1, 1)
    g_h = g.reshape(N, H, HID).transpose(1, 0, 2)

    out = pl.pallas_call(
        _attn_kernel,
        grid=(H, N // BI),
        in_specs=[
            pl.BlockSpec((1, BI, 1), lambda h, ti: (h, ti, 0)),
            pl.BlockSpec((1, 1, N), lambda h, ti: (h, 0, 0)),
            pl.BlockSpec((1, N, 1), lambda h, ti: (h, 0, 0)),
            pl.BlockSpec((1, 1, 1), lambda h, ti: (h, 0, 0)),
            pl.BlockSpec((1, N, HID), lambda h, ti: (h, 0, 0)),
        ],
        out_specs=pl.BlockSpec((1, BI, HID), lambda h, ti: (h, ti, 0)),
        out_shape=jax.ShapeDtypeStruct((H, N, HID), jnp.float32),
    )(ssrc_col, sdst_row, sdst_col, tmax, g_h)
    return out.transpose(1, 0, 2).reshape(N, OUT_F)
